# baseline (speedup 1.0000x reference)
import functools

import jax
import jax.numpy as jnp
from jax.experimental import pallas as pl
from jax.experimental.pallas import tpu as pltpu

INPUT_DIM = 2048
LATENT_DIM = 16384
TOPK = 64
BATCH = 128

LAT_BLK = 1024
N_BLKS = LATENT_DIM // LAT_BLK

_PREC = jax.lax.Precision.DEFAULT


def _select_thresholds(v):
    k = TOPK

    rmax = jnp.max(v, axis=1, keepdims=True)
    lo0 = jnp.zeros((BATCH, 1), jnp.float32)
    hi0 = rmax * jnp.float32(1.000001) + jnp.float32(1e-30)
    cnt0 = jnp.full((BATCH, 1), jnp.int32(LATENT_DIM))

    def body(_, st):
        lo, hi, cnt_lo = st
        mid = (lo + hi) * jnp.float32(0.5)
        cnt = jnp.sum((v >= mid).astype(jnp.int32), axis=1, keepdims=True)
        ge = cnt >= k
        return (jnp.where(ge, mid, lo),
                jnp.where(ge, hi, mid),
                jnp.where(ge, cnt, cnt_lo))

    lo, _, cnt_lo = jax.lax.fori_loop(0, 24, body, (lo0, hi0, cnt0))

    t_bits = jax.lax.bitcast_convert_type(lo, jnp.int32) - 1
    m = jnp.zeros((BATCH, 1), jnp.int32)
    return t_bits, m, jnp.any(cnt_lo != k)


def _select_exact(v):
    k = TOPK
    vb = jax.lax.bitcast_convert_type(v, jnp.int32)
    blo = jnp.zeros((BATCH, 1), jnp.int32)
    bhi = jnp.full((BATCH, 1), jnp.int32(0x7F800000))

    def bbody(_, lohi):
        blo, bhi = lohi
        mid = blo + ((bhi - blo) >> 1)
        cnt = jnp.sum((vb >= mid).astype(jnp.int32), axis=1, keepdims=True)
        ge = cnt >= k
        return (jnp.where(ge, mid, blo), jnp.where(ge, bhi, mid))

    blo, _ = jax.lax.fori_loop(0, 31, bbody, (blo, bhi))
    t_bits = blo

    tie = vb == t_bits
    c_gt = jnp.sum((vb > t_bits).astype(jnp.int32), axis=1, keepdims=True)
    need = k - c_gt

    idx = jax.lax.broadcasted_iota(jnp.int32, (BATCH, LATENT_DIM), 1)

    def tbody(_, lohi):
        lo2, hi2 = lohi
        mid = lo2 + ((hi2 - lo2) >> 1)
        cnt = jnp.sum((tie & (idx < mid)).astype(jnp.int32), axis=1,
                      keepdims=True)
        ge = cnt >= need
        return (jnp.where(ge, lo2, mid), jnp.where(ge, mid, hi2))

    lo2 = jnp.zeros((BATCH, 1), jnp.int32)
    hi2 = jnp.full((BATCH, 1), jnp.int32(LATENT_DIM))
    _, hi2 = jax.lax.fori_loop(0, 14, tbody, (lo2, hi2))
    return t_bits, hi2


def _fused_kernel(x_ref, w_enc_ref, b_ref, w_dec_ref,
                  masked_ref, recon_ref,
                  lat_scr, tb_scr, m_scr):
    i = pl.program_id(0)

    @pl.when(i < N_BLKS)
    def _encode():
        acc = jax.lax.dot_general(
            x_ref[...], w_enc_ref[...], (((1,), (1,)), ((), ())),
            preferred_element_type=jnp.float32, precision=_PREC)
        lat_scr[:, pl.ds(i * LAT_BLK, LAT_BLK)] = jnp.maximum(
            acc + b_ref[...], 0.0)

    @pl.when(i == N_BLKS)
    def _select():
        v = lat_scr[...]
        t_bits, m, unresolved = _select_thresholds(v)
        tb_scr[...] = t_bits
        m_scr[...] = m

        @pl.when(unresolved)
        def _():
            t_bits, m = _select_exact(v)
            tb_scr[...] = t_bits
            m_scr[...] = m

    @pl.when(i >= N_BLKS)
    def _decode():
        j = i - N_BLKS
        v = lat_scr[:, pl.ds(j * LAT_BLK, LAT_BLK)]
        vb = jax.lax.bitcast_convert_type(v, jnp.int32)
        idx = (jax.lax.broadcasted_iota(jnp.int32, (BATCH, LAT_BLK), 1)
               + j * LAT_BLK)
        sel = (vb > tb_scr[...]) | ((vb == tb_scr[...]) & (idx < m_scr[...]))
        mv = v * sel.astype(jnp.float32)
        masked_ref[...] = mv

        @pl.when(i == N_BLKS)
        def _():
            recon_ref[...] = jnp.zeros_like(recon_ref)

        recon_ref[...] += jax.lax.dot_general(
            mv, w_dec_ref[...], (((1,), (1,)), ((), ())),
            preferred_element_type=jnp.float32, precision=_PREC)


@functools.partial(jax.jit, static_argnames=("interpret",))
def kernel(x, W_enc, b_enc, W_dec, interpret=False):
    b2 = b_enc.reshape(1, LATENT_DIM)
    nb = N_BLKS

    masked, recon = pl.pallas_call(
        _fused_kernel,
        grid=(2 * nb,),
        in_specs=[
            pl.BlockSpec((BATCH, INPUT_DIM), lambda i: (0, 0)),
            pl.BlockSpec((LAT_BLK, INPUT_DIM),
                         lambda i: (jnp.minimum(i, nb - 1), 0)),
            pl.BlockSpec((1, LAT_BLK),
                         lambda i: (0, jnp.minimum(i, nb - 1))),
            pl.BlockSpec((INPUT_DIM, LAT_BLK),
                         lambda i: (0, jnp.maximum(i - nb, 0))),
        ],
        out_specs=[
            pl.BlockSpec((BATCH, LAT_BLK),
                         lambda i: (0, jnp.maximum(i - nb, 0))),
            pl.BlockSpec((BATCH, INPUT_DIM), lambda i: (0, 0)),
        ],
        out_shape=[
            jax.ShapeDtypeStruct((BATCH, LATENT_DIM), jnp.float32),
            jax.ShapeDtypeStruct((BATCH, INPUT_DIM), jnp.float32),
        ],
        scratch_shapes=[
            pltpu.VMEM((BATCH, LATENT_DIM), jnp.float32),
            pltpu.VMEM((BATCH, 1), jnp.int32),
            pltpu.VMEM((BATCH, 1), jnp.int32),
        ],
        interpret=interpret,
    )(x, W_enc, b2, W_dec)

    return (recon, masked)


# speedup vs baseline: 1.0423x; 1.0423x over previous
import functools

import jax
import jax.numpy as jnp
from jax.experimental import pallas as pl
from jax.experimental.pallas import tpu as pltpu

INPUT_DIM = 2048
LATENT_DIM = 16384
TOPK = 64
BATCH = 128

LAT_BLK = 1024
N_BLKS = LATENT_DIM // LAT_BLK

_PREC = jax.lax.Precision.DEFAULT


def _select_thresholds(v):
    k = TOPK

    vb = jax.lax.bitcast_convert_type(v, jnp.int32)
    rmax = jnp.max(v, axis=1, keepdims=True)
    lo0 = jnp.zeros((BATCH, 1), jnp.float32)
    hi0 = rmax * jnp.float32(1.000001) + jnp.float32(1e-30)
    cnt0 = jnp.full((BATCH, 1), jnp.int32(LATENT_DIM))

    def cond(st):
        i, _, _, cnt_lo = st
        return (i < 40) & jnp.any(cnt_lo != k)

    def body(st):
        i, lo, hi, cnt_lo = st
        mid = (lo + hi) * jnp.float32(0.5)
        midb = jax.lax.bitcast_convert_type(mid, jnp.int32)
        cnt = LATENT_DIM + jnp.sum((vb - midb) >> 31, axis=1, keepdims=True)
        ge = cnt >= k
        return (i + 1,
                jnp.where(ge, mid, lo),
                jnp.where(ge, hi, mid),
                jnp.where(ge, cnt, cnt_lo))

    _, lo, _, cnt_lo = jax.lax.while_loop(
        cond, body, (jnp.int32(0), lo0, hi0, cnt0))

    t_bits = jax.lax.bitcast_convert_type(lo, jnp.int32) - 1
    m = jnp.zeros((BATCH, 1), jnp.int32)
    return t_bits, m, jnp.any(cnt_lo != k)


def _select_exact(v):
    k = TOPK
    vb = jax.lax.bitcast_convert_type(v, jnp.int32)
    blo = jnp.zeros((BATCH, 1), jnp.int32)
    bhi = jnp.full((BATCH, 1), jnp.int32(0x7F800000))

    def bbody(_, lohi):
        blo, bhi = lohi
        mid = blo + ((bhi - blo) >> 1)
        cnt = jnp.sum((vb >= mid).astype(jnp.int32), axis=1, keepdims=True)
        ge = cnt >= k
        return (jnp.where(ge, mid, blo), jnp.where(ge, bhi, mid))

    blo, _ = jax.lax.fori_loop(0, 31, bbody, (blo, bhi))
    t_bits = blo

    tie = vb == t_bits
    c_gt = jnp.sum((vb > t_bits).astype(jnp.int32), axis=1, keepdims=True)
    need = k - c_gt

    idx = jax.lax.broadcasted_iota(jnp.int32, (BATCH, LATENT_DIM), 1)

    def tbody(_, lohi):
        lo2, hi2 = lohi
        mid = lo2 + ((hi2 - lo2) >> 1)
        cnt = jnp.sum((tie & (idx < mid)).astype(jnp.int32), axis=1,
                      keepdims=True)
        ge = cnt >= need
        return (jnp.where(ge, lo2, mid), jnp.where(ge, mid, hi2))

    lo2 = jnp.zeros((BATCH, 1), jnp.int32)
    hi2 = jnp.full((BATCH, 1), jnp.int32(LATENT_DIM))
    _, hi2 = jax.lax.fori_loop(0, 14, tbody, (lo2, hi2))
    return t_bits, hi2


def _fused_kernel(x_ref, w_enc_ref, b_ref, w_dec_ref,
                  masked_ref, recon_ref,
                  lat_scr, tb_scr, m_scr):
    i = pl.program_id(0)

    @pl.when(i < N_BLKS)
    def _encode():
        acc = jax.lax.dot_general(
            x_ref[...], w_enc_ref[...], (((1,), (1,)), ((), ())),
            preferred_element_type=jnp.float32, precision=_PREC)
        lat_scr[:, pl.ds(i * LAT_BLK, LAT_BLK)] = jnp.maximum(
            acc + b_ref[...], 0.0)

    @pl.when(i == N_BLKS)
    def _select():
        v = lat_scr[...]
        t_bits, m, unresolved = _select_thresholds(v)
        tb_scr[...] = t_bits
        m_scr[...] = m

        @pl.when(unresolved)
        def _():
            t_bits, m = _select_exact(v)
            tb_scr[...] = t_bits
            m_scr[...] = m

    @pl.when(i >= N_BLKS)
    def _decode():
        j = i - N_BLKS
        v = lat_scr[:, pl.ds(j * LAT_BLK, LAT_BLK)]
        vb = jax.lax.bitcast_convert_type(v, jnp.int32)
        idx = (jax.lax.broadcasted_iota(jnp.int32, (BATCH, LAT_BLK), 1)
               + j * LAT_BLK)
        sel = (vb > tb_scr[...]) | ((vb == tb_scr[...]) & (idx < m_scr[...]))
        mv = v * sel.astype(jnp.float32)
        masked_ref[...] = mv

        @pl.when(i == N_BLKS)
        def _():
            recon_ref[...] = jnp.zeros_like(recon_ref)

        recon_ref[...] += jax.lax.dot_general(
            mv, w_dec_ref[...], (((1,), (1,)), ((), ())),
            preferred_element_type=jnp.float32, precision=_PREC)


@functools.partial(jax.jit, static_argnames=("interpret",))
def kernel(x, W_enc, b_enc, W_dec, interpret=False):
    b2 = b_enc.reshape(1, LATENT_DIM)
    nb = N_BLKS

    masked, recon = pl.pallas_call(
        _fused_kernel,
        grid=(2 * nb,),
        in_specs=[
            pl.BlockSpec((BATCH, INPUT_DIM), lambda i: (0, 0)),
            pl.BlockSpec((LAT_BLK, INPUT_DIM),
                         lambda i: (jnp.minimum(i, nb - 1), 0)),
            pl.BlockSpec((1, LAT_BLK),
                         lambda i: (0, jnp.minimum(i, nb - 1))),
            pl.BlockSpec((INPUT_DIM, LAT_BLK),
                         lambda i: (0, jnp.maximum(i - nb, 0))),
        ],
        out_specs=[
            pl.BlockSpec((BATCH, LAT_BLK),
                         lambda i: (0, jnp.maximum(i - nb, 0))),
            pl.BlockSpec((BATCH, INPUT_DIM), lambda i: (0, 0)),
        ],
        out_shape=[
            jax.ShapeDtypeStruct((BATCH, LATENT_DIM), jnp.float32),
            jax.ShapeDtypeStruct((BATCH, INPUT_DIM), jnp.float32),
        ],
        scratch_shapes=[
            pltpu.VMEM((BATCH, LATENT_DIM), jnp.float32),
            pltpu.VMEM((BATCH, 1), jnp.int32),
            pltpu.VMEM((BATCH, 1), jnp.int32),
        ],
        interpret=interpret,
    )(x, W_enc, b2, W_dec)

    return (recon, masked)


# speedup vs baseline: 1.0440x; 1.0016x over previous
import functools

import jax
import jax.numpy as jnp
from jax.experimental import pallas as pl
from jax.experimental.pallas import tpu as pltpu

INPUT_DIM = 2048
LATENT_DIM = 16384
TOPK = 64
BATCH = 128

LAT_BLK = 1024
N_BLKS = LATENT_DIM // LAT_BLK

_PREC = jax.lax.Precision.DEFAULT


def _select_thresholds(v):
    k = TOPK

    vb = jax.lax.bitcast_convert_type(v, jnp.int32)
    rmax = jnp.max(v, axis=1, keepdims=True)
    lo0 = jnp.zeros((BATCH, 1), jnp.float32)
    hi0 = rmax * jnp.float32(1.000001) + jnp.float32(1e-30)
    cnt0 = jnp.full((BATCH, 1), jnp.int32(LATENT_DIM))

    def cond(st):
        i, _, _, cnt_lo = st
        return (i < 40) & jnp.any(cnt_lo != k)

    def step(lo, hi, cnt_lo):
        mid = (lo + hi) * jnp.float32(0.5)
        midb = jax.lax.bitcast_convert_type(mid, jnp.int32)
        cnt = LATENT_DIM + jnp.sum((vb - midb) >> 31, axis=1, keepdims=True)
        ge = cnt >= k
        return (jnp.where(ge, mid, lo),
                jnp.where(ge, hi, mid),
                jnp.where(ge, cnt, cnt_lo))

    def body(st):
        i, lo, hi, cnt_lo = st
        lo, hi, cnt_lo = step(lo, hi, cnt_lo)
        lo, hi, cnt_lo = step(lo, hi, cnt_lo)
        return (i + 2, lo, hi, cnt_lo)

    _, lo, _, cnt_lo = jax.lax.while_loop(
        cond, body, (jnp.int32(0), lo0, hi0, cnt0))

    t_bits = jax.lax.bitcast_convert_type(lo, jnp.int32) - 1
    m = jnp.zeros((BATCH, 1), jnp.int32)
    return t_bits, m, jnp.any(cnt_lo != k)


def _select_exact(v):
    k = TOPK
    vb = jax.lax.bitcast_convert_type(v, jnp.int32)
    blo = jnp.zeros((BATCH, 1), jnp.int32)
    bhi = jnp.full((BATCH, 1), jnp.int32(0x7F800000))

    def bbody(_, lohi):
        blo, bhi = lohi
        mid = blo + ((bhi - blo) >> 1)
        cnt = jnp.sum((vb >= mid).astype(jnp.int32), axis=1, keepdims=True)
        ge = cnt >= k
        return (jnp.where(ge, mid, blo), jnp.where(ge, bhi, mid))

    blo, _ = jax.lax.fori_loop(0, 31, bbody, (blo, bhi))
    t_bits = blo

    tie = vb == t_bits
    c_gt = jnp.sum((vb > t_bits).astype(jnp.int32), axis=1, keepdims=True)
    need = k - c_gt

    idx = jax.lax.broadcasted_iota(jnp.int32, (BATCH, LATENT_DIM), 1)

    def tbody(_, lohi):
        lo2, hi2 = lohi
        mid = lo2 + ((hi2 - lo2) >> 1)
        cnt = jnp.sum((tie & (idx < mid)).astype(jnp.int32), axis=1,
                      keepdims=True)
        ge = cnt >= need
        return (jnp.where(ge, lo2, mid), jnp.where(ge, mid, hi2))

    lo2 = jnp.zeros((BATCH, 1), jnp.int32)
    hi2 = jnp.full((BATCH, 1), jnp.int32(LATENT_DIM))
    _, hi2 = jax.lax.fori_loop(0, 14, tbody, (lo2, hi2))
    return t_bits, hi2


def _fused_kernel(x_ref, w_enc_ref, b_ref, w_dec_ref,
                  masked_ref, recon_ref,
                  lat_scr, tb_scr, m_scr):
    i = pl.program_id(0)

    @pl.when(i < N_BLKS)
    def _encode():
        acc = jax.lax.dot_general(
            x_ref[...], w_enc_ref[...], (((1,), (1,)), ((), ())),
            preferred_element_type=jnp.float32, precision=_PREC)
        lat_scr[:, pl.ds(i * LAT_BLK, LAT_BLK)] = jnp.maximum(
            acc + b_ref[...], 0.0)

    @pl.when(i == N_BLKS)
    def _select():
        v = lat_scr[...]
        t_bits, m, unresolved = _select_thresholds(v)
        tb_scr[...] = t_bits
        m_scr[...] = m

        @pl.when(unresolved)
        def _():
            t_bits, m = _select_exact(v)
            tb_scr[...] = t_bits
            m_scr[...] = m

    @pl.when(i >= N_BLKS)
    def _decode():
        j = i - N_BLKS
        v = lat_scr[:, pl.ds(j * LAT_BLK, LAT_BLK)]
        vb = jax.lax.bitcast_convert_type(v, jnp.int32)
        idx = (jax.lax.broadcasted_iota(jnp.int32, (BATCH, LAT_BLK), 1)
               + j * LAT_BLK)
        sel = (vb > tb_scr[...]) | ((vb == tb_scr[...]) & (idx < m_scr[...]))
        mv = v * sel.astype(jnp.float32)
        masked_ref[...] = mv

        @pl.when(i == N_BLKS)
        def _():
            recon_ref[...] = jnp.zeros_like(recon_ref)

        recon_ref[...] += jax.lax.dot_general(
            mv, w_dec_ref[...], (((1,), (1,)), ((), ())),
            preferred_element_type=jnp.float32, precision=_PREC)


@functools.partial(jax.jit, static_argnames=("interpret",))
def kernel(x, W_enc, b_enc, W_dec, interpret=False):
    b2 = b_enc.reshape(1, LATENT_DIM)
    nb = N_BLKS

    masked, recon = pl.pallas_call(
        _fused_kernel,
        grid=(2 * nb,),
        in_specs=[
            pl.BlockSpec((BATCH, INPUT_DIM), lambda i: (0, 0)),
            pl.BlockSpec((LAT_BLK, INPUT_DIM),
                         lambda i: (jnp.minimum(i, nb - 1), 0)),
            pl.BlockSpec((1, LAT_BLK),
                         lambda i: (0, jnp.minimum(i, nb - 1))),
            pl.BlockSpec((INPUT_DIM, LAT_BLK),
                         lambda i: (0, jnp.maximum(i - nb, 0))),
        ],
        out_specs=[
            pl.BlockSpec((BATCH, LAT_BLK),
                         lambda i: (0, jnp.maximum(i - nb, 0))),
            pl.BlockSpec((BATCH, INPUT_DIM), lambda i: (0, 0)),
        ],
        out_shape=[
            jax.ShapeDtypeStruct((BATCH, LATENT_DIM), jnp.float32),
            jax.ShapeDtypeStruct((BATCH, INPUT_DIM), jnp.float32),
        ],
        scratch_shapes=[
            pltpu.VMEM((BATCH, LATENT_DIM), jnp.float32),
            pltpu.VMEM((BATCH, 1), jnp.int32),
            pltpu.VMEM((BATCH, 1), jnp.int32),
        ],
        interpret=interpret,
    )(x, W_enc, b2, W_dec)

    return (recon, masked)
